# baseline (speedup 1.0000x reference)
import jax
import jax.numpy as jnp
from jax import lax
from jax.experimental import pallas as pl
from jax.experimental.pallas import tpu as pltpu
from jax.experimental.pallas import tpu_sc as plsc

NC = 2
NS = 16
L = 16
NW = NC * NS

BATCH = 4096
SEQ = 50
N_TOKENS = BATCH * SEQ
N_SLOTS = 128
D = 64

B_PER_W = N_TOKENS // NW
ROWS_PER_W = BATCH // NW
CHUNK = 128
N_CHUNKS = B_PER_W // CHUNK
NBUF = 2
N_EXT = 2 * N_SLOTS


def _sc_body(ids_hbm, first_hbm, embt_hbm, out_hbm,
             ids_v, first_v, stg_v, tbl_v, blk_v, ssem):
    wid = lax.axis_index("s") * NC + lax.axis_index("c")
    base = wid * B_PER_W

    pltpu.sync_copy(first_hbm.at[pl.ds(0, L)], first_v)
    lanes = lax.iota(jnp.int32, L)
    lo = first_v[...] - lanes
    hi = lo + N_SLOTS

    pltpu.sync_copy(embt_hbm, stg_v)
    zeros = lanes * 0

    def build(d, carry):
        rep = plsc.load_gather(stg_v, [zeros + d, zeros])
        for k in range(N_SLOTS // L):
            tbl_v[pl.ds(d * N_EXT + k * L, L)] = stg_v[d, pl.ds(k * L, L)]
            tbl_v[pl.ds(d * N_EXT + N_SLOTS + k * L, L)] = rep
        return carry

    lax.fori_loop(0, D, build, 0)

    for d in range(D):
        rep_d = plsc.load_gather(tbl_v, [zeros + (d * N_EXT + N_SLOTS)])
        for b in range(NBUF):
            for l0 in range(CHUNK // L):
                blk_v[b, d // 8, 0, d % 8, pl.ds(l0 * L, L)] = rep_d

    pltpu.sync_copy(ids_hbm.at[pl.ds(wid * ROWS_PER_W, ROWS_PER_W)], ids_v)

    def store_handle(c, b):
        k = base // CHUNK + c
        return pltpu.make_async_copy(
            blk_v.at[b], out_hbm.at[pl.ds(0, D // 8), pl.ds(k, 1)], ssem)

    def run(c, mask):
        b = c % NBUF

        @pl.when(c >= NBUF)
        def _():
            store_handle(c - NBUF, b).wait()

        for l0 in range(CHUNK // L):
            s = c * CHUNK + l0 * L + lanes
            r = lax.div(s, jnp.int32(SEQ))
            t = plsc.load_gather(ids_v, [r, s - r * SEQ])
            ok = (t >= lo) & (t < hi)
            sidx = jnp.where(ok, t - lo, N_SLOTS + (t & (N_SLOTS - 1)))
            hit = jnp.any(ok)
            bitp = jnp.int32(1) << (b * (CHUNK // L) + l0)
            dirty = (mask & bitp) != 0

            @pl.when(hit | dirty)
            def _():
                fidx = sidx
                for d in range(D):
                    blk_v[b, d // 8, 0, d % 8, pl.ds(l0 * L, L)] = (
                        plsc.load_gather(tbl_v, [fidx]))
                    if d + 1 < D:
                        fidx = fidx + N_EXT

            mask = jnp.where(hit, mask | bitp, mask & ~bitp)

        store_handle(c, b).start()
        return mask

    lax.fori_loop(0, N_CHUNKS, run, jnp.int32(0))
    for c in range(N_CHUNKS - NBUF, N_CHUNKS):
        store_handle(c, c % NBUF).wait()


@jax.jit
def _lookup(ids, input_ids, embt):
    mesh = plsc.VectorSubcoreMesh(core_axis_name="c", subcore_axis_name="s",
                                  num_cores=NC, num_subcores=NS)
    f = pl.kernel(
        _sc_body,
        out_type=jax.ShapeDtypeStruct((8, N_TOKENS // CHUNK, 8, CHUNK),
                                      jnp.float32),
        mesh=mesh,
        scratch_types=[
            pltpu.VMEM((ROWS_PER_W, SEQ), jnp.int32),
            pltpu.VMEM((L,), jnp.int32),
            pltpu.VMEM((D, N_SLOTS), jnp.float32),
            pltpu.VMEM((D * N_EXT,), jnp.float32),
            pltpu.VMEM((NBUF, D // 8, 1, 8, CHUNK), jnp.float32),
            pltpu.SemaphoreType.DMA,
        ],
        compiler_params=pltpu.CompilerParams(use_tc_tiling_on_sc=False,
                                             needs_layout_passes=False),
    )
    out4 = f(ids, input_ids, embt)
    return out4.transpose(1, 3, 0, 2).reshape(N_TOKENS, D)


def kernel(prompt_token_ids, input_ids, emb_weight):
    return _lookup(prompt_token_ids, input_ids, emb_weight.T)


# speedup vs baseline: 1.5019x; 1.5019x over previous
import jax
import jax.numpy as jnp
from jax import lax
from jax.experimental import pallas as pl
from jax.experimental.pallas import tpu as pltpu
from jax.experimental.pallas import tpu_sc as plsc

NC = 2
NS = 16
L = 16
NW = NC * NS

BATCH = 4096
SEQ = 50
N_TOKENS = BATCH * SEQ
N_SLOTS = 128
D = 64

B_PER_W = N_TOKENS // NW
ROWS_PER_W = BATCH // NW
CHUNK = 128
N_CHUNKS = B_PER_W // CHUNK
NBUF = 4
N_EXT = 2 * N_SLOTS


def _sc_body(ids_hbm, first_hbm, embt_hbm, out_hbm,
             ids_v, idx_v, first_v, stg_v, tbl_v, blk_v, ssem, isem):
    wid = lax.axis_index("s") * NC + lax.axis_index("c")
    base = wid * B_PER_W

    ids_cp = pltpu.async_copy(
        ids_hbm.at[pl.ds(wid * ROWS_PER_W, ROWS_PER_W)], ids_v, isem)

    pltpu.sync_copy(first_hbm.at[pl.ds(0, L)], first_v)
    lanes = lax.iota(jnp.int32, L)
    lo = first_v[...] - lanes
    hi = lo + N_SLOTS

    pltpu.sync_copy(embt_hbm, stg_v)
    zeros = lanes * 0

    def build(d, carry):
        rep = plsc.load_gather(stg_v, [zeros + d, zeros])
        for k in range(N_SLOTS // L):
            tbl_v[pl.ds(d * N_EXT + k * L, L)] = stg_v[d, pl.ds(k * L, L)]
            tbl_v[pl.ds(d * N_EXT + N_SLOTS + k * L, L)] = rep
        return carry

    lax.fori_loop(0, D, build, 0)

    for d in range(D):
        rep_d = plsc.load_gather(tbl_v, [zeros + (d * N_EXT + N_SLOTS)])
        for b in range(NBUF):
            for l0 in range(CHUNK // L):
                blk_v[b, d // 8, 0, d % 8, pl.ds(l0 * L, L)] = rep_d

    ids_cp.wait()

    def remap(g, carry):
        s = g * L + lanes
        r = lax.div(s, jnp.int32(SEQ))
        t = plsc.load_gather(ids_v, [r, s - r * SEQ])
        ok = (t >= lo) & (t < hi)
        idx_v[pl.ds(g * L, L)] = jnp.where(
            ok, t - lo, N_SLOTS + (t & (N_SLOTS - 1)))
        return carry

    lax.fori_loop(0, B_PER_W // L, remap, 0)

    def store_handle(c, b):
        k = base // CHUNK + c
        return pltpu.make_async_copy(
            blk_v.at[b], out_hbm.at[pl.ds(0, D // 8), pl.ds(k, 1)], ssem)

    def run(c, mask):
        b = c % NBUF

        @pl.when(c >= NBUF)
        def _():
            store_handle(c - NBUF, b).wait()

        for l0 in range(CHUNK // L):
            sidx = idx_v[pl.ds(c * CHUNK + l0 * L, L)]
            hit = jnp.any(sidx < N_SLOTS)
            bitp = jnp.int32(1) << (b * (CHUNK // L) + l0)
            dirty = (mask & bitp) != 0

            @pl.when(hit | dirty)
            def _():
                fidx = sidx
                for d in range(D):
                    blk_v[b, d // 8, 0, d % 8, pl.ds(l0 * L, L)] = (
                        plsc.load_gather(tbl_v, [fidx]))
                    if d + 1 < D:
                        fidx = fidx + N_EXT

            mask = jnp.where(hit, mask | bitp, mask & ~bitp)

        store_handle(c, b).start()
        return mask

    lax.fori_loop(0, N_CHUNKS, run, jnp.int32(0))
    for c in range(N_CHUNKS - NBUF, N_CHUNKS):
        store_handle(c, c % NBUF).wait()


@jax.jit
def _lookup(ids, input_ids, embt):
    mesh = plsc.VectorSubcoreMesh(core_axis_name="c", subcore_axis_name="s",
                                  num_cores=NC, num_subcores=NS)
    f = pl.kernel(
        _sc_body,
        out_type=jax.ShapeDtypeStruct((8, N_TOKENS // CHUNK, 8, CHUNK),
                                      jnp.float32),
        mesh=mesh,
        scratch_types=[
            pltpu.VMEM((ROWS_PER_W, SEQ), jnp.int32),
            pltpu.VMEM((B_PER_W,), jnp.int32),
            pltpu.VMEM((L,), jnp.int32),
            pltpu.VMEM((D, N_SLOTS), jnp.float32),
            pltpu.VMEM((D * N_EXT,), jnp.float32),
            pltpu.VMEM((NBUF, D // 8, 1, 8, CHUNK), jnp.float32),
            pltpu.SemaphoreType.DMA,
            pltpu.SemaphoreType.DMA,
        ],
        compiler_params=pltpu.CompilerParams(use_tc_tiling_on_sc=False,
                                             needs_layout_passes=False),
    )
    out4 = f(ids, input_ids, embt)
    return out4.transpose(1, 3, 0, 2).reshape(N_TOKENS, D)


def kernel(prompt_token_ids, input_ids, emb_weight):
    return _lookup(prompt_token_ids, input_ids, emb_weight.T)
